# baseline (speedup 1.0000x reference)
import functools

import jax
import jax.numpy as jnp
from jax import lax
from jax.experimental import pallas as pl
from jax.experimental.pallas import tpu as pltpu
from jax.experimental.pallas import tpu_sc as plsc

_NUM_FEATURES = 256
_NUM_PIDS = 100000
_OIM_SCALAR = 30.0
_IGNORE_INDEX = 5554
_BATCH = 256

_BN = 10000
_K = _NUM_PIDS // _BN

_LOG2E = 1.4426950408889634
_LN2 = 0.6931471805599453


def _make_sc_gather():
    info = plsc.get_sparse_core_info()
    nc, ns = info.num_cores, info.num_subcores
    nworkers = 16
    b_per_w = _BATCH // nworkers
    mesh = plsc.VectorSubcoreMesh(core_axis_name="c", subcore_axis_name="s")

    @functools.partial(
        pl.kernel,
        mesh=mesh,
        out_type=[
            jax.ShapeDtypeStruct((_BATCH,), jnp.float32),
            jax.ShapeDtypeStruct((_BATCH, _NUM_FEATURES), jnp.float32),
        ],
        scratch_types=[
            pltpu.VMEM((b_per_w,), jnp.int32),
            pltpu.VMEM((b_per_w,), jnp.int32),
            pltpu.VMEM((b_per_w,), jnp.int32),
            pltpu.VMEM((b_per_w,), jnp.float32),
            pltpu.VMEM((b_per_w, _NUM_FEATURES), jnp.float32),
            pltpu.SemaphoreType.DMA,
        ],
    )
    def sc_gather(roi_hbm, labels_hbm, lut_hbm, valid_out, rows_out,
                  roi_v, safe_v, lbl_v, val_v, rows_v, sem):
        wid = lax.axis_index("s") * nc + lax.axis_index("c")

        @pl.when(wid < nworkers)
        def _work():
            base = wid * b_per_w
            pltpu.sync_copy(roi_hbm.at[pl.ds(base, b_per_w)], roi_v)
            t = roi_v[...] - 1
            inds = t >= 0
            safe_v[...] = jnp.where(inds, t, 0)
            pltpu.async_copy(labels_hbm.at[safe_v], lbl_v, sem).wait()
            lab = lbl_v[...]
            val_v[...] = jnp.where(inds & (lab != _IGNORE_INDEX), 1.0, 0.0)
            pltpu.sync_copy(val_v, valid_out.at[pl.ds(base, b_per_w)])
            pltpu.async_copy(lut_hbm.at[lbl_v], rows_v, sem).wait()
            pltpu.sync_copy(rows_v, rows_out.at[pl.ds(base, b_per_w)])

    return sc_gather


_sc_gather_cache = []


def _get_sc_gather():
    if not _sc_gather_cache:
        _sc_gather_cache.append(_make_sc_gather())
    return _sc_gather_cache[0]


def _tc_body(x_ref, lut_ref, logz_ref, m_s, s_s):
    k = pl.program_id(0)

    @pl.when(k == 0)
    def _init():
        m_s[...] = jnp.full((_BATCH, 1), -jnp.inf, jnp.float32)
        s_s[...] = jnp.zeros((_BATCH, 1), jnp.float32)

    xbf = (x_ref[...] * (_OIM_SCALAR * _LOG2E)).astype(jnp.bfloat16)
    proj2 = lax.dot_general(
        xbf, lut_ref[...].astype(jnp.bfloat16),
        (((1,), (1,)), ((), ())),
        preferred_element_type=jnp.float32,
    )
    bm = jnp.max(proj2, axis=1, keepdims=True)
    m_new = jnp.maximum(m_s[...], bm)
    e2 = jnp.exp2(proj2 - m_new)
    bsum = lax.dot_general(
        e2, jnp.ones((_BN, 1), jnp.float32),
        (((1,), (0,)), ((), ())),
        preferred_element_type=jnp.float32,
    )
    s_s[...] = s_s[...] * jnp.exp2(m_s[...] - m_new) + bsum
    m_s[...] = m_new

    @pl.when(k == _K - 1)
    def _finish():
        logz_ref[...] = (m_s[...] + jnp.log2(s_s[...])) * _LN2


_tc_logz = pl.pallas_call(
    _tc_body,
    grid=(_K,),
    in_specs=[
        pl.BlockSpec((_BATCH, _NUM_FEATURES), lambda k: (0, 0)),
        pl.BlockSpec((_BN, _NUM_FEATURES), lambda k: (k, 0)),
    ],
    out_specs=pl.BlockSpec((_BATCH, 1), lambda k: (0, 0)),
    out_shape=jax.ShapeDtypeStruct((_BATCH, 1), jnp.float32),
    scratch_shapes=[
        pltpu.VMEM((_BATCH, 1), jnp.float32),
        pltpu.VMEM((_BATCH, 1), jnp.float32),
    ],
)


def _epi_body(x_ref, glut_ref, valid_ref, logz_ref, out_ref):
    picked = jnp.sum(x_ref[...] * glut_ref[...], axis=1,
                     keepdims=True) * _OIM_SCALAR
    nll = logz_ref[...] - picked
    v = valid_ref[...]
    total = jnp.sum(v * nll)
    cnt = jnp.sum(v)
    out_ref[...] = (total / jnp.maximum(cnt, 1.0)).reshape(1, 1)


_tc_epi = pl.pallas_call(
    _epi_body,
    out_shape=jax.ShapeDtypeStruct((1, 1), jnp.float32),
)


def kernel(inputs, roi_label, lut, labels):
    valid, glut = _get_sc_gather()(roi_label.astype(jnp.int32), labels, lut)
    logz = _tc_logz(inputs, lut)
    out = _tc_epi(inputs, glut, valid.reshape(_BATCH, 1), logz)
    return out[0, 0]


# speedup vs baseline: 1.4338x; 1.4338x over previous
import functools

import jax
import jax.numpy as jnp
from jax import lax
from jax.experimental import pallas as pl
from jax.experimental.pallas import tpu as pltpu
from jax.experimental.pallas import tpu_sc as plsc

_NUM_FEATURES = 256
_NUM_PIDS = 100000
_OIM_SCALAR = 30.0
_IGNORE_INDEX = 5554
_BATCH = 256

_BN = 10000
_K = _NUM_PIDS // _BN

_LOG2E = 1.4426950408889634
_LN2 = 0.6931471805599453


def _make_sc_gather():
    info = plsc.get_sparse_core_info()
    nc, ns = info.num_cores, info.num_subcores
    nworkers = 16
    b_per_w = _BATCH // nworkers
    mesh = plsc.VectorSubcoreMesh(core_axis_name="c", subcore_axis_name="s")

    @functools.partial(
        pl.kernel,
        mesh=mesh,
        out_type=[
            jax.ShapeDtypeStruct((_BATCH,), jnp.float32),
            jax.ShapeDtypeStruct((_BATCH, _NUM_FEATURES), jnp.float32),
        ],
        scratch_types=[
            pltpu.VMEM((b_per_w,), jnp.int32),
            pltpu.VMEM((b_per_w,), jnp.int32),
            pltpu.VMEM((b_per_w,), jnp.int32),
            pltpu.VMEM((b_per_w,), jnp.float32),
            pltpu.VMEM((b_per_w, _NUM_FEATURES), jnp.float32),
            pltpu.SemaphoreType.DMA,
        ],
    )
    def sc_gather(roi_hbm, labels_hbm, lut_hbm, valid_out, rows_out,
                  roi_v, safe_v, lbl_v, val_v, rows_v, sem):
        wid = lax.axis_index("s") * nc + lax.axis_index("c")

        @pl.when(wid < nworkers)
        def _work():
            base = wid * b_per_w
            pltpu.sync_copy(roi_hbm.at[pl.ds(base, b_per_w)], roi_v)
            t = roi_v[...] - 1
            inds = t >= 0
            safe_v[...] = jnp.where(inds, t, 0)
            pltpu.async_copy(labels_hbm.at[safe_v], lbl_v, sem).wait()
            lab = lbl_v[...]
            val_v[...] = jnp.where(inds & (lab != _IGNORE_INDEX), 1.0, 0.0)
            pltpu.sync_copy(val_v, valid_out.at[pl.ds(base, b_per_w)])
            pltpu.async_copy(lut_hbm.at[lbl_v], rows_v, sem).wait()
            pltpu.sync_copy(rows_v, rows_out.at[pl.ds(base, b_per_w)])

    return sc_gather


_sc_gather_cache = []


def _get_sc_gather():
    if not _sc_gather_cache:
        _sc_gather_cache.append(_make_sc_gather())
    return _sc_gather_cache[0]


def _tc_body(x_ref, lut_ref, logz_ref, m_s, s_s):
    k = pl.program_id(0)

    @pl.when(k == 0)
    def _init():
        m_s[...] = jnp.full((_BATCH, 1), -jnp.inf, jnp.float32)
        s_s[...] = jnp.zeros((_BATCH, 1), jnp.float32)

    xbf = (x_ref[...] * (_OIM_SCALAR * _LOG2E)).astype(jnp.bfloat16)
    proj2 = lax.dot_general(
        xbf, lut_ref[...].astype(jnp.bfloat16),
        (((1,), (1,)), ((), ())),
        preferred_element_type=jnp.float32,
    ).astype(jnp.bfloat16)
    bm = jnp.max(proj2, axis=1, keepdims=True).astype(jnp.float32)
    m_new = jnp.maximum(m_s[...], bm)
    d = proj2 - m_new.astype(jnp.bfloat16)
    s_s[...] = s_s[...] * jnp.exp2(m_s[...] - m_new) + jnp.sum(
        jnp.exp2(d.astype(jnp.float32)), axis=1, keepdims=True)
    m_s[...] = m_new

    @pl.when(k == _K - 1)
    def _finish():
        logz_ref[...] = (m_s[...] + jnp.log2(s_s[...])) * _LN2


_tc_logz = pl.pallas_call(
    _tc_body,
    grid=(_K,),
    in_specs=[
        pl.BlockSpec((_BATCH, _NUM_FEATURES), lambda k: (0, 0)),
        pl.BlockSpec((_BN, _NUM_FEATURES), lambda k: (k, 0)),
    ],
    out_specs=pl.BlockSpec((_BATCH, 1), lambda k: (0, 0)),
    out_shape=jax.ShapeDtypeStruct((_BATCH, 1), jnp.float32),
    scratch_shapes=[
        pltpu.VMEM((_BATCH, 1), jnp.float32),
        pltpu.VMEM((_BATCH, 1), jnp.float32),
    ],
)


def _epi_body(x_ref, glut_ref, valid_ref, logz_ref, out_ref):
    picked = jnp.sum(x_ref[...] * glut_ref[...], axis=1,
                     keepdims=True) * _OIM_SCALAR
    nll = logz_ref[...] - picked
    v = valid_ref[...]
    total = jnp.sum(v * nll)
    cnt = jnp.sum(v)
    out_ref[...] = (total / jnp.maximum(cnt, 1.0)).reshape(1, 1)


_tc_epi = pl.pallas_call(
    _epi_body,
    out_shape=jax.ShapeDtypeStruct((1, 1), jnp.float32),
)


def kernel(inputs, roi_label, lut, labels):
    valid, glut = _get_sc_gather()(roi_label.astype(jnp.int32), labels, lut)
    logz = _tc_logz(inputs, lut)
    out = _tc_epi(inputs, glut, valid.reshape(_BATCH, 1), logz)
    return out[0, 0]
